# Initial kernel scaffold; baseline (speedup 1.0000x reference)
import jax
import jax.numpy as jnp
from jax.experimental import pallas as pl


def kernel(x):
    raise NotImplementedError("write your pallas kernel here")


# speedup vs baseline: 33.5440x; 33.5440x over previous
import functools

import jax
import jax.numpy as jnp
from jax import lax
from jax.experimental import pallas as pl
from jax.experimental.pallas import tpu as pltpu
from jax.experimental.pallas import tpu_sc as plsc

ROWS = 64
ROW_LEN = 32768
OUT_K = 256
LANES = 16
NV = OUT_K // LANES
NCHUNK = ROW_LEN // OUT_K
NWORKERS = 32
ROWS_PER_W = ROWS // NWORKERS


def _vsort_desc(v):
    k, _ = plsc.sort_key_val(v, v, descending=True)
    return k


def _vrev(v):
    return lax.rev(v, (0,))


def _bitonic_merge_desc(s):
    m = len(s)
    if m == 1:
        return [_vsort_desc(s[0])]
    h = m // 2
    hi = [jnp.maximum(s[i], s[i + h]) for i in range(h)]
    lo = [jnp.minimum(s[i], s[i + h]) for i in range(h)]
    return _bitonic_merge_desc(hi) + _bitonic_merge_desc(lo)


def _merge_sorted_desc(a, b):
    s = a + [_vrev(x) for x in reversed(b)]
    return _bitonic_merge_desc(s)


def _topk_merge(a, b):
    rb = [_vrev(x) for x in reversed(b)]
    hi = [jnp.maximum(x, y) for x, y in zip(a, rb)]
    return _bitonic_merge_desc(hi)


def _sort_chunk(vs):
    runs = [[_vsort_desc(v)] for v in vs]
    while len(runs) > 1:
        runs = [_merge_sorted_desc(runs[i], runs[i + 1])
                for i in range(0, len(runs), 2)]
    return runs[0]


def _make_kernel():
    mesh = plsc.VectorSubcoreMesh(core_axis_name="c", subcore_axis_name="s")

    @functools.partial(
        pl.kernel,
        mesh=mesh,
        out_type=jax.ShapeDtypeStruct((ROWS, OUT_K), jnp.float32),
        scratch_types=[
            pltpu.VMEM((ROW_LEN,), jnp.float32),
            pltpu.VMEM((OUT_K,), jnp.float32),
        ],
        compiler_params=pltpu.CompilerParams(needs_layout_passes=False),
    )
    def topk_rows(x_hbm, out_hbm, row_v, out_v):
        wid = lax.axis_index("s") * 2 + lax.axis_index("c")

        def do_row(t, _):
            row = wid * ROWS_PER_W + t
            pltpu.sync_copy(x_hbm.at[row], row_v)

            def load_chunk(j):
                return [row_v[pl.ds(j * OUT_K + i * LANES, LANES)]
                        for i in range(NV)]

            acc = _sort_chunk(load_chunk(0))

            def fold(j, acc):
                cur = _sort_chunk(load_chunk(j))
                return tuple(_topk_merge(list(acc), cur))

            acc = lax.fori_loop(1, NCHUNK, fold, tuple(acc))

            for i in range(NV):
                out_v[pl.ds(i * LANES, LANES)] = acc[i]
            pltpu.sync_copy(out_v, out_hbm.at[row])
            return 0

        lax.fori_loop(0, ROWS_PER_W, do_row, 0)

    return topk_rows


_topk_kernel = _make_kernel()


def kernel(x):
    return _topk_kernel(x)


# speedup vs baseline: 34.1845x; 1.0191x over previous
import functools

import jax
import jax.numpy as jnp
from jax import lax
from jax.experimental import pallas as pl
from jax.experimental.pallas import tpu as pltpu
from jax.experimental.pallas import tpu_sc as plsc

ROWS = 64
ROW_LEN = 32768
OUT_K = 256
LANES = 16
NV = OUT_K // LANES
NCHUNK = ROW_LEN // OUT_K
NWORKERS = 32
ROWS_PER_W = ROWS // NWORKERS


def _vsort(v, desc):
    k, _ = plsc.sort_key_val(v, v, descending=desc)
    return k


def _bitonic_merge(s, desc):
    m = len(s)
    if m == 1:
        return [_vsort(s[0], desc)]
    h = m // 2
    if desc:
        a = [jnp.maximum(s[i], s[i + h]) for i in range(h)]
        b = [jnp.minimum(s[i], s[i + h]) for i in range(h)]
    else:
        a = [jnp.minimum(s[i], s[i + h]) for i in range(h)]
        b = [jnp.maximum(s[i], s[i + h]) for i in range(h)]
    return _bitonic_merge(a, desc) + _bitonic_merge(b, desc)


def _build_sorted(vs, desc):
    if len(vs) == 1:
        return [_vsort(vs[0], desc)]
    h = len(vs) // 2
    a = _build_sorted(vs[:h], True)
    b = _build_sorted(vs[h:], False)
    return _bitonic_merge(a + b, desc)


def _topk_merge(acc_desc, chunk_asc):
    hi = [jnp.maximum(x, y) for x, y in zip(acc_desc, chunk_asc)]
    return _bitonic_merge(hi, True)


def _make_kernel():
    mesh = plsc.VectorSubcoreMesh(core_axis_name="c", subcore_axis_name="s")

    @functools.partial(
        pl.kernel,
        mesh=mesh,
        out_type=jax.ShapeDtypeStruct((ROWS, OUT_K), jnp.float32),
        scratch_types=[
            pltpu.VMEM((ROW_LEN,), jnp.float32),
            pltpu.VMEM((OUT_K,), jnp.float32),
        ],
        compiler_params=pltpu.CompilerParams(needs_layout_passes=False),
    )
    def topk_rows(x_hbm, out_hbm, row_v, out_v):
        wid = lax.axis_index("s") * 2 + lax.axis_index("c")

        def do_row(t, _):
            row = wid * ROWS_PER_W + t
            pltpu.sync_copy(x_hbm.at[row], row_v)

            def load_chunk(j):
                return [row_v[pl.ds(j * OUT_K + i * LANES, LANES)]
                        for i in range(NV)]

            acc = _build_sorted(load_chunk(0), True)

            def fold(j, acc):
                cur = _build_sorted(load_chunk(j), False)
                return tuple(_topk_merge(list(acc), cur))

            acc = lax.fori_loop(1, NCHUNK, fold, tuple(acc))

            for i in range(NV):
                out_v[pl.ds(i * LANES, LANES)] = acc[i]
            pltpu.sync_copy(out_v, out_hbm.at[row])
            return 0

        lax.fori_loop(0, ROWS_PER_W, do_row, 0)

    return topk_rows


_topk_kernel = _make_kernel()


def kernel(x):
    return _topk_kernel(x)


# speedup vs baseline: 45.0969x; 1.3192x over previous
import functools

import jax
import jax.numpy as jnp
from jax import lax
from jax.experimental import pallas as pl
from jax.experimental.pallas import tpu as pltpu
from jax.experimental.pallas import tpu_sc as plsc

ROWS = 64
ROW_LEN = 32768
OUT_K = 256
LANES = 16
NV = OUT_K // LANES
NCHUNK = ROW_LEN // OUT_K
NWORKERS = 32
ROWS_PER_W = ROWS // NWORKERS


def _vsort(v, desc):
    k, _ = plsc.sort_key_val(v, v, descending=desc)
    return k


def _bitonic_merge(s, desc):
    m = len(s)
    if m == 1:
        return [_vsort(s[0], desc)]
    h = m // 2
    if desc:
        a = [jnp.maximum(s[i], s[i + h]) for i in range(h)]
        b = [jnp.minimum(s[i], s[i + h]) for i in range(h)]
    else:
        a = [jnp.minimum(s[i], s[i + h]) for i in range(h)]
        b = [jnp.maximum(s[i], s[i + h]) for i in range(h)]
    return _bitonic_merge(a, desc) + _bitonic_merge(b, desc)


def _build_sorted(vs, desc):
    if len(vs) == 1:
        return [_vsort(vs[0], desc)]
    h = len(vs) // 2
    a = _build_sorted(vs[:h], True)
    b = _build_sorted(vs[h:], False)
    return _bitonic_merge(a + b, desc)


def _topk_merge(acc_desc, chunk_asc):
    hi = [jnp.maximum(x, y) for x, y in zip(acc_desc, chunk_asc)]
    return _bitonic_merge(hi, True)


def _make_kernel():
    mesh = plsc.VectorSubcoreMesh(core_axis_name="c", subcore_axis_name="s")

    @functools.partial(
        pl.kernel,
        mesh=mesh,
        out_type=jax.ShapeDtypeStruct((ROWS, OUT_K), jnp.float32),
        scratch_types=[
            pltpu.VMEM((ROW_LEN,), jnp.float32),
            pltpu.VMEM((ROW_LEN + OUT_K,), jnp.float32),
            pltpu.VMEM((OUT_K,), jnp.float32),
        ],
        compiler_params=pltpu.CompilerParams(needs_layout_passes=False),
    )
    def topk_rows(x_hbm, out_hbm, row_v, cand_v, out_v):
        wid = lax.axis_index("s") * 2 + lax.axis_index("c")
        iota = lax.iota(jnp.int32, LANES)
        ninf = jnp.full((LANES,), -jnp.inf, jnp.float32)

        def do_row(t, _):
            row = wid * ROWS_PER_W + t
            pltpu.sync_copy(x_hbm.at[row], row_v)

            def p1(b, m):
                return tuple(
                    jnp.maximum(m[i], row_v[pl.ds(b * OUT_K + i * LANES, LANES)])
                    for i in range(NV))

            m = lax.fori_loop(
                1, NCHUNK, p1,
                tuple(row_v[pl.ds(i * LANES, LANES)] for i in range(NV)))
            m = list(m)
            while len(m) > 1:
                h = len(m) // 2
                m = [jnp.minimum(m[i], m[i + h]) for i in range(h)]
            thr = jnp.min(m[0])

            def p2(b, off):
                vals, masks, cums, pops = [], [], [], []
                for i in range(NV):
                    v = row_v[pl.ds(b * OUT_K + i * LANES, LANES)]
                    mask = v >= thr
                    vals.append(v)
                    masks.append(mask)
                    cums.append(plsc.cumsum(mask.astype(jnp.int32)))
                    pops.append(plsc.all_reduce_population_count(mask))
                pref = off
                for i in range(NV):
                    idx = pref + cums[i] - 1
                    plsc.store_scatter(cand_v, [idx], vals[i], mask=masks[i])
                    pref = pref + pops[i]
                tot = pops
                while len(tot) > 1:
                    tot = [tot[i] + tot[i + len(tot) // 2]
                           for i in range(len(tot) // 2)]
                return off + tot[0]

            off = lax.fori_loop(0, NCHUNK, p2, jnp.zeros((LANES,), jnp.int32))

            for i in range(NV):
                plsc.store_scatter(cand_v, [off + (iota + i * LANES)], ninf)

            count = jnp.max(off)
            nch = (count + (OUT_K - 1)) // OUT_K

            def fold(j, acc):
                cur = _build_sorted(
                    [cand_v[pl.ds(j * OUT_K + i * LANES, LANES)]
                     for i in range(NV)], False)
                return tuple(_topk_merge(list(acc), cur))

            acc = lax.fori_loop(0, nch, fold, tuple(ninf for _ in range(NV)))

            for i in range(NV):
                out_v[pl.ds(i * LANES, LANES)] = acc[i]
            pltpu.sync_copy(out_v, out_hbm.at[row])
            return 0

        lax.fori_loop(0, ROWS_PER_W, do_row, 0)

    return topk_rows


_topk_kernel = _make_kernel()


def kernel(x):
    return _topk_kernel(x)
